# Initial kernel scaffold; baseline (speedup 1.0000x reference)
import jax
import jax.numpy as jnp
from jax.experimental import pallas as pl


def kernel(src, seg, W_word, W_pos, W_seg):
    raise NotImplementedError("write your pallas kernel here")


# speedup vs baseline: 1.0914x; 1.0914x over previous
import functools

import jax
import jax.numpy as jnp
from jax import lax
from jax.experimental import pallas as pl
from jax.experimental.pallas import tpu as pltpu
from jax.experimental.pallas import tpu_sc as plsc

B, S, H, VOCAB = 64, 512, 768, 100000
PPW = 16
HS = H // 16


def _seg_bcast(sgf_p):
    sv = sgf_p[...]
    return [sv.at[jnp.full((16,), r, jnp.int32)].get(mode="promise_in_bounds")
            for r in range(PPW)]


def _add_posseg(rows_p, segb, posw, dloc):
    def hbody(h, c):
        off = pl.multiple_of(h * 16, 16)
        dh = dloc[pl.ds(off, 16)]
        for r in range(PPW):
            sl = pl.ds(off, 16)
            rows_p[r, sl] = rows_p[r, sl] + (posw[r, sl] + segb[r] * dh)
        return c

    lax.fori_loop(0, HS, hbody, 0)


def _body(src, segf, wword, posa, d, out, posw, dloc, idxv, sgfv, rows,
          gsem, ssem, isem):
    info = plsc.get_sparse_core_info()
    nc = info.num_cores
    wid = lax.axis_index("s") * nc + lax.axis_index("c")
    pbase = wid * PPW

    pltpu.sync_copy(posa.at[pl.ds(pbase, PPW)], posw)
    pltpu.sync_copy(d, dloc)

    def bbody(b, c):
        pltpu.async_copy(src.at[b, pl.ds(pbase, PPW)], idxv, isem)
        pltpu.async_copy(segf.at[b, pl.ds(pbase, PPW)], sgfv, isem).wait()
        pltpu.make_async_copy(src.at[b, pl.ds(pbase, PPW)], idxv, isem).wait()
        gcopy = pltpu.async_copy(wword.at[idxv], rows, gsem)
        segb = _seg_bcast(sgfv)
        gcopy.wait()
        _add_posseg(rows, segb, posw, dloc)
        pltpu.async_copy(rows, out.at[b, pl.ds(pbase, PPW)], ssem).wait()
        return c

    lax.fori_loop(0, B, bbody, 0)


_mesh = plsc.VectorSubcoreMesh(core_axis_name="c", subcore_axis_name="s")

_sc_call = functools.partial(
    pl.kernel,
    out_type=jax.ShapeDtypeStruct((B, S, H), jnp.float32),
    mesh=_mesh,
    scratch_types=[
        pltpu.VMEM((PPW, H), jnp.float32),
        pltpu.VMEM((H,), jnp.float32),
        pltpu.VMEM((PPW,), jnp.int32),
        pltpu.VMEM((PPW,), jnp.float32),
        pltpu.VMEM((PPW, H), jnp.float32),
        pltpu.SemaphoreType.DMA,
        pltpu.SemaphoreType.DMA,
        pltpu.SemaphoreType.DMA,
    ],
)(_body)


@jax.jit
def kernel(src, seg, W_word, W_pos, W_seg):
    s0 = W_seg[0]
    dd = W_seg[1] - s0
    posa = W_pos + s0
    segf = seg.astype(jnp.float32)
    return _sc_call(src, segf, W_word, posa, dd)


# speedup vs baseline: 2.2746x; 2.0840x over previous
import functools

import jax
import jax.numpy as jnp
from jax import lax
from jax.experimental import pallas as pl
from jax.experimental.pallas import tpu as pltpu
from jax.experimental.pallas import tpu_sc as plsc

B, S, H, VOCAB = 64, 512, 768, 100000
PPW = 16
HS = H // 16
NB = 4


def _seg_bcast(sgf_p):
    sv = sgf_p[...]
    return [sv.at[jnp.full((16,), r, jnp.int32)].get(mode="promise_in_bounds")
            for r in range(PPW)]


def _add_posseg(rows_p, segb, posw, dloc):
    def hbody(h, c):
        off = pl.multiple_of(h * 16, 16)
        dh = dloc[pl.ds(off, 16)]
        for r in range(PPW):
            sl = pl.ds(off, 16)
            rows_p[r, sl] = rows_p[r, sl] + (posw[r, sl] + segb[r] * dh)
        return c

    lax.fori_loop(0, HS, hbody, 0)


def _body(src, segf, wword, posa, d, out, posw, dloc, idx, sgf, rows, *sems):
    gsem = sems[0:NB]
    ssem = sems[NB:2 * NB]
    isem = sems[2 * NB:3 * NB]
    info = plsc.get_sparse_core_info()
    nc = info.num_cores
    wid = lax.axis_index("s") * nc + lax.axis_index("c")
    pbase = wid * PPW
    psl = pl.ds(pbase, PPW)

    pltpu.sync_copy(posa.at[psl], posw)
    pltpu.sync_copy(d, dloc)

    def load_inputs(b, p):
        pltpu.async_copy(src.at[b, psl], idx.at[p], isem[p])
        pltpu.async_copy(segf.at[b, psl], sgf.at[p], isem[p])

    def wait_inputs(b, p):
        pltpu.make_async_copy(src.at[b, psl], idx.at[p], isem[p]).wait()
        pltpu.make_async_copy(segf.at[b, psl], sgf.at[p], isem[p]).wait()

    for p in range(NB):
        load_inputs(p, p)

    def process_q(bq, q):
        pltpu.make_async_copy(wword.at[idx.at[q]], rows.at[q], gsem[q]).wait()
        segb = _seg_bcast(sgf.at[q])
        pl.when(bq + NB < B)(lambda: load_inputs(bq + NB, q))
        _add_posseg(rows.at[q], segb, posw, dloc)
        pltpu.async_copy(rows.at[q], out.at[bq, psl], ssem[q])

    def ibody(i, c):
        for p in range(NB):
            b = i * NB + p
            q = (p - 1) % NB

            def wait_scatter(p=p):
                pltpu.make_async_copy(
                    rows.at[p], out.at[0, psl], ssem[p]).wait()

            def start_p(b=b, p=p):
                wait_inputs(b, p)
                pltpu.async_copy(wword.at[idx.at[p]], rows.at[p], gsem[p])

            pl.when(i >= 1)(wait_scatter)
            start_p()
            if p == 0:
                pl.when(i >= 1)(lambda b=b, q=q: process_q(b - 1, q))
            else:
                process_q(b - 1, q)
        return c

    lax.fori_loop(0, B // NB, ibody, 0)

    process_q(B - 1, NB - 1)
    for p in range(NB):
        pltpu.make_async_copy(rows.at[p], out.at[0, psl], ssem[p]).wait()


_mesh = plsc.VectorSubcoreMesh(core_axis_name="c", subcore_axis_name="s")

_sc_call = functools.partial(
    pl.kernel,
    out_type=jax.ShapeDtypeStruct((B, S, H), jnp.float32),
    mesh=_mesh,
    scratch_types=[
        pltpu.VMEM((PPW, H), jnp.float32),
        pltpu.VMEM((H,), jnp.float32),
        pltpu.VMEM((NB, PPW), jnp.int32),
        pltpu.VMEM((NB, PPW), jnp.float32),
        pltpu.VMEM((NB, PPW, H), jnp.float32),
    ] + [pltpu.SemaphoreType.DMA] * (3 * NB),
)(_body)


@jax.jit
def kernel(src, seg, W_word, W_pos, W_seg):
    s0 = W_seg[0]
    dd = W_seg[1] - s0
    posa = W_pos + s0
    segf = seg.astype(jnp.float32)
    return _sc_call(src, segf, W_word, posa, dd)


# speedup vs baseline: 2.3242x; 1.0218x over previous
import functools

import jax
import jax.numpy as jnp
from jax import lax
from jax.experimental import pallas as pl
from jax.experimental.pallas import tpu as pltpu
from jax.experimental.pallas import tpu_sc as plsc

B, S, H, VOCAB = 64, 512, 768, 100000
PPW = 16
HS = H // 16
NB = 4
LEAD = 2


def _seg_bcast(sgf_p):
    sv = sgf_p[...].astype(jnp.float32)
    return [sv.at[jnp.full((16,), r, jnp.int32)].get(mode="promise_in_bounds")
            for r in range(PPW)]


def _add_posseg(rows_p, segb, posw, dloc):
    def hbody(h, c):
        off = pl.multiple_of(h * 16, 16)
        dh = dloc[pl.ds(off, 16)]
        for r in range(PPW):
            sl = pl.ds(off, 16)
            rows_p[r, sl] = rows_p[r, sl] + (posw[r, sl] + segb[r] * dh)
        return c

    lax.fori_loop(0, HS, hbody, 0)


def _body(src, seg, wword, wpos, wseg, out,
          posw, dloc, wsg, idx, sgf, rows, *sems):
    gsem = sems[0:NB]
    ssem = sems[NB:2 * NB]
    isem = sems[2 * NB:3 * NB]
    info = plsc.get_sparse_core_info()
    nc = info.num_cores
    wid = lax.axis_index("s") * nc + lax.axis_index("c")
    pbase = wid * PPW
    psl = pl.ds(pbase, PPW)

    pltpu.sync_copy(wpos.at[psl], posw)
    pltpu.sync_copy(wseg, wsg)

    def prep_h(h, c):
        off = pl.multiple_of(h * 16, 16)
        sl = pl.ds(off, 16)
        s0h = wsg[0, sl]
        dloc[sl] = wsg[1, sl] - s0h
        for r in range(PPW):
            posw[r, sl] = posw[r, sl] + s0h
        return c

    lax.fori_loop(0, HS, prep_h, 0)

    def load_inputs(b, p):
        pltpu.async_copy(src.at[b, psl], idx.at[p], isem[p])
        pltpu.async_copy(seg.at[b, psl], sgf.at[p], isem[p])

    def wait_inputs(b, p):
        pltpu.make_async_copy(src.at[b, psl], idx.at[p], isem[p]).wait()
        pltpu.make_async_copy(seg.at[b, psl], sgf.at[p], isem[p]).wait()

    for t in range(NB):
        load_inputs(t, t)

    def start_gather(t, p):
        wait_inputs(t, p)
        pltpu.async_copy(wword.at[idx.at[p]], rows.at[p], gsem[p])

    def process(bq, q):
        pltpu.make_async_copy(wword.at[idx.at[q]], rows.at[q], gsem[q]).wait()
        segb = _seg_bcast(sgf.at[q])
        pl.when(bq + NB < B)(lambda: load_inputs(bq + NB, q))
        _add_posseg(rows.at[q], segb, posw, dloc)
        pltpu.async_copy(rows.at[q], out.at[bq, psl], ssem[q])

    def ibody(i, c):
        for p in range(NB):
            t = i * NB + p
            q = (p - LEAD) % NB

            def free_rows(p=p):
                pltpu.make_async_copy(
                    rows.at[p], out.at[0, psl], ssem[p]).wait()

            pl.when(i >= 1)(free_rows)
            start_gather(t, p)
            if p < LEAD:
                pl.when(i >= 1)(lambda t=t, q=q: process(t - LEAD, q))
            else:
                process(t - LEAD, q)
        return c

    lax.fori_loop(0, B // NB, ibody, 0)

    for k in range(LEAD):
        bq = B - LEAD + k
        process(bq, bq % NB)
    for p in range(NB):
        pltpu.make_async_copy(rows.at[p], out.at[0, psl], ssem[p]).wait()


_mesh = plsc.VectorSubcoreMesh(core_axis_name="c", subcore_axis_name="s")

_sc_call = functools.partial(
    pl.kernel,
    out_type=jax.ShapeDtypeStruct((B, S, H), jnp.float32),
    mesh=_mesh,
    scratch_types=[
        pltpu.VMEM((PPW, H), jnp.float32),
        pltpu.VMEM((H,), jnp.float32),
        pltpu.VMEM((2, H), jnp.float32),
        pltpu.VMEM((NB, PPW), jnp.int32),
        pltpu.VMEM((NB, PPW), jnp.int32),
        pltpu.VMEM((NB, PPW, H), jnp.float32),
    ] + [pltpu.SemaphoreType.DMA] * (3 * NB),
)(_body)


@jax.jit
def kernel(src, seg, W_word, W_pos, W_seg):
    return _sc_call(src, seg, W_word, W_pos, W_seg)


# speedup vs baseline: 2.7571x; 1.1862x over previous
import functools

import jax
import jax.numpy as jnp
from jax import lax
from jax.experimental import pallas as pl
from jax.experimental.pallas import tpu as pltpu
from jax.experimental.pallas import tpu_sc as plsc

B, S, H, VOCAB = 64, 512, 768, 100000
PPW = 16
HS = H // 16
NB = 4
LEAD = 2


def _seg_bcast(sgf_p):
    sv = sgf_p[...].astype(jnp.float32)
    return [sv.at[jnp.full((16,), r, jnp.int32)].get(mode="promise_in_bounds")
            for r in range(PPW)]


def _add_posseg(rows_p, segb, posw, dloc):
    def hbody(h, c):
        off = pl.multiple_of(h * 16, 16)
        dh = dloc[pl.ds(off, 16)]
        for r in range(PPW):
            sl = pl.ds(off, 16)
            plsc.addupdate(rows_p.at[r, sl], posw[r, sl] + segb[r] * dh)
        return c

    lax.fori_loop(0, HS, hbody, 0)


def _body(src, seg, wword, wpos, wseg, out,
          posw, dloc, wsg, idx, sgf, rows, *sems):
    gsem = sems[0:NB]
    ssem = sems[NB:2 * NB]
    isem = sems[2 * NB:3 * NB]
    info = plsc.get_sparse_core_info()
    nc = info.num_cores
    wid = lax.axis_index("s") * nc + lax.axis_index("c")
    pbase = wid * PPW
    psl = pl.ds(pbase, PPW)

    pltpu.sync_copy(wpos.at[psl], posw)
    pltpu.sync_copy(wseg, wsg)

    def prep_h(h, c):
        off = pl.multiple_of(h * 16, 16)
        sl = pl.ds(off, 16)
        s0h = wsg[0, sl]
        dloc[sl] = wsg[1, sl] - s0h
        for r in range(PPW):
            posw[r, sl] = posw[r, sl] + s0h
        return c

    lax.fori_loop(0, HS, prep_h, 0)

    def load_inputs(b, p):
        pltpu.async_copy(src.at[b, psl], idx.at[p], isem[p])
        pltpu.async_copy(seg.at[b, psl], sgf.at[p], isem[p])

    def wait_inputs(b, p):
        pltpu.make_async_copy(src.at[b, psl], idx.at[p], isem[p]).wait()
        pltpu.make_async_copy(seg.at[b, psl], sgf.at[p], isem[p]).wait()

    for t in range(NB):
        load_inputs(t, t)

    def start_gather(t, p):
        wait_inputs(t, p)
        pltpu.async_copy(wword.at[idx.at[p]], rows.at[p], gsem[p])

    def process(bq, q):
        pltpu.make_async_copy(wword.at[idx.at[q]], rows.at[q], gsem[q]).wait()
        segb = _seg_bcast(sgf.at[q])
        pl.when(bq + NB < B)(lambda: load_inputs(bq + NB, q))
        _add_posseg(rows.at[q], segb, posw, dloc)
        pltpu.async_copy(rows.at[q], out.at[bq, psl], ssem[q])

    def ibody(i, c):
        for p in range(NB):
            t = i * NB + p
            q = (p - LEAD) % NB

            def free_rows(p=p):
                pltpu.make_async_copy(
                    rows.at[p], out.at[0, psl], ssem[p]).wait()

            pl.when(i >= 1)(free_rows)
            start_gather(t, p)
            if p < LEAD:
                pl.when(i >= 1)(lambda t=t, q=q: process(t - LEAD, q))
            else:
                process(t - LEAD, q)
        return c

    lax.fori_loop(0, B // NB, ibody, 0)

    for k in range(LEAD):
        bq = B - LEAD + k
        process(bq, bq % NB)
    for p in range(NB):
        pltpu.make_async_copy(rows.at[p], out.at[0, psl], ssem[p]).wait()


_mesh = plsc.VectorSubcoreMesh(core_axis_name="c", subcore_axis_name="s")

_sc_call = functools.partial(
    pl.kernel,
    out_type=jax.ShapeDtypeStruct((B, S, H), jnp.float32),
    mesh=_mesh,
    scratch_types=[
        pltpu.VMEM((PPW, H), jnp.float32),
        pltpu.VMEM((H,), jnp.float32),
        pltpu.VMEM((2, H), jnp.float32),
        pltpu.VMEM((NB, PPW), jnp.int32),
        pltpu.VMEM((NB, PPW), jnp.int32),
        pltpu.VMEM((NB, PPW, H), jnp.float32),
    ] + [pltpu.SemaphoreType.DMA] * (3 * NB),
)(_body)


@jax.jit
def kernel(src, seg, W_word, W_pos, W_seg):
    return _sc_call(src, seg, W_word, W_pos, W_seg)
